# Initial kernel scaffold; baseline (speedup 1.0000x reference)
import jax
import jax.numpy as jnp
from jax.experimental import pallas as pl


def kernel(input_tensor, Wq, bq, Wk, bk, Wv, bv, Wd, bd, ln1_g, ln1_b, W1, b1, W2, b2, ln2_g, ln2_b):
    raise NotImplementedError("write your pallas kernel here")


# speedup vs baseline: 26.9910x; 26.9910x over previous
import functools
import math

import numpy as np
import jax
import jax.numpy as jnp
from jax.experimental import pallas as pl
from jax.experimental.pallas import tpu as pltpu

_L = 4096
_D = 768
_TOPK = 41
_F0 = 819
_NF = 1280
_FB = 256
_RB = 256
_TILE = 16
_BF = jnp.bfloat16
_F32 = jnp.float32


def _np_split_bf16(a):
    hi = a.astype(jnp.bfloat16)
    lo = (a - hi.astype(np.float32)).astype(jnp.bfloat16)
    return hi, lo


def _build_consts():
    t = np.arange(_L, dtype=np.float64)
    f = (_F0 + np.arange(_NF, dtype=np.float64))[:, None]
    ang = 2.0 * np.pi * f * t[None, :] / _L
    ec = np.cos(ang).astype(np.float32)
    es = (-np.sin(ang)).astype(np.float32)
    ech, ecl = _np_split_bf16(ec)
    esh, esl = _np_split_bf16(es)
    i32 = np.arange(32, dtype=np.float64)
    i128 = np.arange(128, dtype=np.float64)
    e32c = np.cos(2 * np.pi * np.outer(i32, i32) / 32).astype(np.float32)
    e32s = np.sin(2 * np.pi * np.outer(i32, i32) / 32).astype(np.float32)
    twc = np.cos(2 * np.pi * np.outer(i128, i32) / _L).astype(np.float32)
    tws = np.sin(2 * np.pi * np.outer(i128, i32) / _L).astype(np.float32)
    e128c = np.cos(2 * np.pi * np.outer(i128, i128) / 128).astype(np.float32)
    e128s = np.sin(2 * np.pi * np.outer(i128, i128) / 128).astype(np.float32)
    return ech, ecl, esh, esl, e32c, e32s, twc, tws, e128c, e128s


_CONSTS = _build_consts()


def _dot(a, b):
    return jnp.dot(a, b, preferred_element_type=jnp.float32)


def _proj_body(x_ref, wq_ref, wk_ref, qh_ref, ql_ref, kh_ref, kl_ref):
    xb = x_ref[...].astype(_BF)
    q = _dot(xb, wq_ref[...])
    k = _dot(xb, wk_ref[...])
    qh = q.astype(_BF)
    kh = k.astype(_BF)
    qh_ref[...] = qh
    ql_ref[...] = (q - qh.astype(_F32)).astype(_BF)
    kh_ref[...] = kh
    kl_ref[...] = (k - kh.astype(_F32)).astype(_BF)


def _spec_body(qh_ref, ql_ref, kh_ref, kl_ref,
               ech_ref, ecl_ref, esh_ref, esl_ref,
               e32c_ref, e32s_ref, twc_ref, tws_ref, e128c_ref, e128s_ref,
               dly_ref, w_ref, s_re_ref, s_im_ref):
    b = pl.program_id(0)

    @pl.when(b == 0)
    def _init():
        s_re_ref[...] = jnp.zeros((32, 128), _F32)
        s_im_ref[...] = jnp.zeros((32, 128), _F32)

    qh, ql = qh_ref[...], ql_ref[...]
    kh, kl = kh_ref[...], kl_ref[...]
    ech, ecl = ech_ref[...], ecl_ref[...]
    esh, esl = esh_ref[...], esl_ref[...]

    def mm3(ahi, alo, bhi, blo):
        return _dot(ahi, bhi) + _dot(ahi, blo) + _dot(alo, bhi)

    qfr = mm3(ech, ecl, qh, ql)
    qfi = mm3(esh, esl, qh, ql)
    kfr = mm3(ech, ecl, kh, kl)
    kfi = mm3(esh, esl, kh, kl)

    sre = jnp.sum((qfr * kfr + qfi * kfi).reshape(2, 128, _D), axis=2)
    sim = jnp.sum((qfi * kfr - qfr * kfi).reshape(2, 128, _D), axis=2)
    s_re_ref[pl.ds(2 * b, 2), :] = sre
    s_im_ref[pl.ds(2 * b, 2), :] = sim

    @pl.when(b == (_NF // _FB) - 1)
    def _finale():
        row = jax.lax.broadcasted_iota(jnp.int32, (32, 128), 0)
        lane = jax.lax.broadcasted_iota(jnp.int32, (32, 128), 1)
        jflat = row * 128 + lane
        beta = np.float32(1.0 / (_L * _D))
        alpha = jnp.where(jflat <= (2047 - _F0), 2.0 * beta,
                          jnp.where(jflat == (2048 - _F0), beta, 0.0))
        are = s_re_ref[...] * alpha
        aim = s_im_ref[...] * alpha * jnp.where(jflat == (2048 - _F0), 0.0, 1.0)

        def shift819(a):
            r6 = jnp.roll(jnp.roll(a, 6, axis=0), 51, axis=1)
            r7 = jnp.roll(jnp.roll(a, 7, axis=0), 51, axis=1)
            return jnp.where(lane >= 51, r6, r7)

        cre = shift819(are)
        cim = shift819(aim)

        dn = (((0,), (0,)), ((), ()))
        e32c, e32s = e32c_ref[...], e32s_ref[...]
        gre = (jax.lax.dot_general(cre, e32c, dn, preferred_element_type=_F32)
               - jax.lax.dot_general(cim, e32s, dn, preferred_element_type=_F32))
        gim = (jax.lax.dot_general(cre, e32s, dn, preferred_element_type=_F32)
               + jax.lax.dot_general(cim, e32c, dn, preferred_element_type=_F32))
        twc, tws = twc_ref[...], tws_ref[...]
        hre = gre * twc - gim * tws
        him = gre * tws + gim * twc
        e128c, e128s = e128c_ref[...], e128s_ref[...]
        mv = (jax.lax.dot_general(e128c, hre, dn, preferred_element_type=_F32)
              - jax.lax.dot_general(e128s, him, dn, preferred_element_type=_F32))
        tidx = (32 * jax.lax.broadcasted_iota(jnp.int32, (128, 32), 0)
                + jax.lax.broadcasted_iota(jnp.int32, (128, 32), 1))

        orow = jax.lax.broadcasted_iota(jnp.int32, (8, 128), 0)
        olane = jax.lax.broadcasted_iota(jnp.int32, (8, 128), 1)
        wvals = jnp.full((8, 128), -jnp.inf, _F32)
        dvals = jnp.zeros((8, 128), jnp.int32)
        for i in range(_TOPK):
            m = jnp.max(mv)
            d = jnp.min(jnp.where(mv == m, tidx, jnp.int32(2 ** 30)))
            sel = (orow == 0) & (olane == i)
            wvals = jnp.where(sel, m, wvals)
            dvals = jnp.where(sel, d, dvals)
            mv = jnp.where(tidx == d, -jnp.inf, mv)
        valid = (orow == 0) & (olane < _TOPK)
        mx = jnp.max(jnp.where(valid, wvals, -jnp.inf))
        e = jnp.where(valid, jnp.exp(wvals - mx), 0.0)
        w_ref[...] = e / jnp.sum(e)
        dly_ref[...] = dvals


def _out_body(x_ref, wv_ref, bv_ref, dly_ref, w_ref,
              wd_ref, bd_ref, g1_ref, bg1_ref, w1_ref, bf1_ref,
              w2_ref, bf2_ref, g2_ref, bg2_ref,
              out_ref, vpad_ref):
    p = pl.program_id(0)
    j = pl.program_id(1)
    xb = x_ref[...]

    @pl.when(p == 0)
    def _compute_v():
        vb = _dot(xb.astype(_BF), wv_ref[...]) + bv_ref[...]
        vb3 = vb.reshape(_RB // 8, 8, _D)
        vpad_ref[pl.ds(j * (_RB // 8), _RB // 8), :, :] = vb3

        @pl.when(j == 0)
        def _pad():
            vpad_ref[pl.ds(_L // 8, _RB // 8), :, :] = vb3

    @pl.when(p == 1)
    def _combine_and_out():
        r0 = j * _RB
        k8s = []
        shifts = []
        ws = []
        for i in range(_TOPK):
            d = dly_ref[0, i]
            s = r0 + d
            s = jnp.where(s >= _L, s - _L, s)
            k8s.append(jax.lax.shift_right_logical(s, 3))
            dlo = jax.lax.bitwise_and(s, 7)
            shifts.append(jnp.where(dlo == 0, 0, 24 - dlo))
            ws.append(w_ref[0, i])
        rows = []
        for tt in range(_RB // _TILE):
            acc = jnp.zeros((_TILE, _D), _F32)
            for i in range(_TOPK):
                t24 = vpad_ref[pl.ds(k8s[i] + 2 * tt, 3), :, :].reshape(24, _D)
                rolled = pltpu.roll(t24, shifts[i], 0)
                acc = acc + ws[i] * rolled[0:_TILE, :]
            rows.append(acc)
        ctx = jnp.concatenate(rows, axis=0)

        h = _dot(ctx.astype(_BF), wd_ref[...]) + bd_ref[...] + xb

        def ln(z, g, bb):
            mu = jnp.mean(z, axis=-1, keepdims=True)
            var = jnp.mean((z - mu) ** 2, axis=-1, keepdims=True)
            return (z - mu) / jnp.sqrt(var + 1e-8) * g + bb

        y = ln(h, g1_ref[...], bg1_ref[...])
        f = _dot(y.astype(_BF), w1_ref[...]) + bf1_ref[...]
        f = f * 0.5 * (1.0 + jax.lax.erf(f / np.float32(math.sqrt(2.0))))
        f2 = _dot(f.astype(_BF), w2_ref[...]) + bf2_ref[...]
        out_ref[...] = ln(f2 + y, g2_ref[...], bg2_ref[...])


def kernel(input_tensor, Wq, bq, Wk, bk, Wv, bv, Wd, bd, ln1_g, ln1_b,
           W1, b1, W2, b2, ln2_g, ln2_b):
    x = input_tensor.reshape(_L, _D)
    ech, ecl, esh, esl, e32c, e32s, twc, tws, e128c, e128s = _CONSTS

    full = lambda shape: pl.BlockSpec(shape, lambda *_: (0,) * len(shape))

    qh, ql, kh, kl = pl.pallas_call(
        _proj_body,
        grid=(8,),
        in_specs=[
            pl.BlockSpec((512, _D), lambda i: (i, 0)),
            full((_D, _D)),
            full((_D, _D)),
        ],
        out_specs=[pl.BlockSpec((512, _D), lambda i: (i, 0))] * 4,
        out_shape=[jax.ShapeDtypeStruct((_L, _D), _BF)] * 4,
    )(x, Wq.astype(_BF), Wk.astype(_BF))

    nsteps = _NF // _FB
    dly, wsm = pl.pallas_call(
        _spec_body,
        grid=(nsteps,),
        in_specs=[full((_L, _D))] * 4 + [
            pl.BlockSpec((_FB, _L), lambda b: (b, 0)),
            pl.BlockSpec((_FB, _L), lambda b: (b, 0)),
            pl.BlockSpec((_FB, _L), lambda b: (b, 0)),
            pl.BlockSpec((_FB, _L), lambda b: (b, 0)),
            full((32, 32)), full((32, 32)),
            full((128, 32)), full((128, 32)),
            full((128, 128)), full((128, 128)),
        ],
        out_specs=[full((8, 128)), full((8, 128))],
        out_shape=[jax.ShapeDtypeStruct((8, 128), jnp.int32),
                   jax.ShapeDtypeStruct((8, 128), _F32)],
        scratch_shapes=[pltpu.VMEM((32, 128), _F32),
                        pltpu.VMEM((32, 128), _F32)],
    )(qh, ql, kh, kl, ech, ecl, esh, esl,
      e32c, e32s, twc, tws, e128c, e128s)

    row = lambda a: a.reshape(1, _D)
    smem = pl.BlockSpec(memory_space=pltpu.SMEM)
    out = pl.pallas_call(
        _out_body,
        grid=(2, _L // _RB),
        in_specs=[
            pl.BlockSpec((_RB, _D), lambda p, j: (j, 0)),
            full((_D, _D)),
            full((1, _D)),
            smem,
            smem,
            full((_D, _D)),
            full((1, _D)),
            full((1, _D)), full((1, _D)),
            full((_D, _D)), full((1, _D)),
            full((_D, _D)), full((1, _D)),
            full((1, _D)), full((1, _D)),
        ],
        out_specs=pl.BlockSpec((_RB, _D), lambda p, j: (j, 0)),
        out_shape=jax.ShapeDtypeStruct((_L, _D), _F32),
        scratch_shapes=[pltpu.VMEM(((_L + _RB) // 8, 8, _D), _F32)],
    )(x, Wv.astype(_BF), row(bv), dly, wsm,
      Wd.astype(_BF), row(bd), row(ln1_g), row(ln1_b),
      W1.astype(_BF), row(b1), W2.astype(_BF), row(b2),
      row(ln2_g), row(ln2_b))

    return out.reshape(1, _L, _D)
